# baseline (speedup 1.0000x reference)
import functools

import jax
import jax.numpy as jnp
from jax import lax
from jax.experimental import pallas as pl
from jax.experimental.pallas import tpu as pltpu
from jax.experimental.pallas import tpu_sc as plsc

_B = 16384 * 200
_D = 64
_NC = 2
_NS = 16
_NW = _NC * _NS
_BPW = _B // _NW
_CHUNK = 512
_STEPS = _BPW // _CHUNK


def _make_gather():
    mesh = plsc.VectorSubcoreMesh(core_axis_name="c", subcore_axis_name="s")

    @functools.partial(
        pl.kernel,
        mesh=mesh,
        out_type=jax.ShapeDtypeStruct((_B, _D), jnp.float32),
        scratch_types=[
            pltpu.VMEM((_CHUNK,), jnp.int32),
            pltpu.VMEM((_CHUNK, _D), jnp.float32),
            pltpu.SemaphoreType.DMA,
        ],
        compiler_params=pltpu.CompilerParams(use_tc_tiling_on_sc=False),
    )
    def k(idx_hbm, table_hbm, out_hbm, idx_v, rows_v, sem):
        wid = lax.axis_index("s") * _NC + lax.axis_index("c")
        base = wid * _BPW

        def body(g, carry):
            off = base + g * _CHUNK
            pltpu.sync_copy(idx_hbm.at[pl.ds(off, _CHUNK)], idx_v)
            pltpu.async_copy(table_hbm.at[idx_v], rows_v, sem).wait()
            pltpu.sync_copy(rows_v, out_hbm.at[pl.ds(off, _CHUNK)])
            return carry

        lax.fori_loop(0, _STEPS, body, 0)

    return k


_gather = _make_gather()


@jax.jit
def kernel(input_ids, weight):
    flat = input_ids.reshape(-1).astype(jnp.int32)
    out = _gather(flat, weight)
    return out.reshape(input_ids.shape + (_D,))


# speedup vs baseline: 1.0756x; 1.0756x over previous
import functools

import jax
import jax.numpy as jnp
from jax import lax
from jax.experimental import pallas as pl
from jax.experimental.pallas import tpu as pltpu
from jax.experimental.pallas import tpu_sc as plsc

_B = 16384 * 200
_D = 64
_NC = 2
_NS = 16
_NW = _NC * _NS
_BPW = _B // _NW
_CHUNK = 400
_STEPS = _BPW // _CHUNK
_NBUF = 4
_GROUPS = _STEPS // _NBUF


def _make_gather():
    mesh = plsc.VectorSubcoreMesh(core_axis_name="c", subcore_axis_name="s")

    @functools.partial(
        pl.kernel,
        mesh=mesh,
        out_type=jax.ShapeDtypeStruct((_B, _D), jnp.float32),
        scratch_types=[
            pltpu.VMEM((_NBUF, _CHUNK), jnp.int32),
            pltpu.VMEM((_NBUF, _CHUNK, _D), jnp.float32),
            pltpu.SemaphoreType.DMA((_NBUF,)),
            pltpu.SemaphoreType.DMA((_NBUF,)),
            pltpu.SemaphoreType.DMA((_NBUF,)),
        ],
        compiler_params=pltpu.CompilerParams(use_tc_tiling_on_sc=False),
    )
    def k(idx_hbm, table_hbm, out_hbm, idx_v, rows_v, sem_i, sem_g, sem_o):
        wid = lax.axis_index("s") * _NC + lax.axis_index("c")
        base = wid * _BPW

        def start_idx(s, b):
            pltpu.async_copy(
                idx_hbm.at[pl.ds(base + s * _CHUNK, _CHUNK)],
                idx_v.at[b], sem_i.at[b])

        def wait_idx(b):
            pltpu.make_async_copy(
                idx_hbm.at[pl.ds(base, _CHUNK)], idx_v.at[b],
                sem_i.at[b]).wait()

        def start_gather(b):
            pltpu.async_copy(table_hbm.at[idx_v.at[b]], rows_v.at[b],
                             sem_g.at[b])

        def wait_gather(b):
            pltpu.make_async_copy(table_hbm.at[idx_v.at[b]], rows_v.at[b],
                                  sem_g.at[b]).wait()

        def start_out(s, b):
            pltpu.async_copy(
                rows_v.at[b],
                out_hbm.at[pl.ds(base + s * _CHUNK, _CHUNK)], sem_o.at[b])

        def wait_out(b):
            pltpu.make_async_copy(
                rows_v.at[b], out_hbm.at[pl.ds(base, _CHUNK)],
                sem_o.at[b]).wait()

        for b in range(_NBUF):
            start_idx(b, b)
        wait_idx(0)
        start_gather(0)
        for b in range(1, _NBUF):
            wait_idx(b)
            start_gather(b)
            wait_gather(b - 1)
            start_out(b - 1, b - 1)
            start_idx(b - 1 + _NBUF, b - 1)

        def body(g, carry):
            for b in range(_NBUF):
                s = g * _NBUF + b
                pb = (b - 1) % _NBUF
                wait_out(b)
                wait_idx(b)
                start_gather(b)
                wait_gather(pb)
                start_out(s - 1, pb)

                @pl.when(s - 1 + _NBUF < _STEPS)
                def _():
                    start_idx(s - 1 + _NBUF, pb)
            return carry

        lax.fori_loop(1, _GROUPS, body, 0)

        last = _NBUF - 1
        wait_gather(last)
        start_out(_STEPS - 1, last)
        for b in range(_NBUF):
            wait_out(b)

    return k


_gather = _make_gather()


@jax.jit
def kernel(input_ids, weight):
    flat = input_ids.reshape(-1).astype(jnp.int32)
    out = _gather(flat, weight)
    return out.reshape(input_ids.shape + (_D,))


# speedup vs baseline: 1.0764x; 1.0007x over previous
import functools

import jax
import jax.numpy as jnp
from jax import lax
from jax.experimental import pallas as pl
from jax.experimental.pallas import tpu as pltpu
from jax.experimental.pallas import tpu_sc as plsc

_R = 16384
_C = 200
_D = 64
_NC = 2
_NS = 16
_NW = _NC * _NS
_RPW = _R // _NW
_RPS = 2
_STEPS = _RPW // _RPS
_NBUF = 4
_GROUPS = _STEPS // _NBUF


def _make_gather():
    mesh = plsc.VectorSubcoreMesh(core_axis_name="c", subcore_axis_name="s")

    @functools.partial(
        pl.kernel,
        mesh=mesh,
        out_type=jax.ShapeDtypeStruct((_R, _C, _D), jnp.float32),
        scratch_types=[
            pltpu.VMEM((_NBUF, _RPS, _C), jnp.int32),
            pltpu.VMEM((_NBUF, _RPS, _C, _D), jnp.float32),
            pltpu.SemaphoreType.DMA((_NBUF,)),
            pltpu.SemaphoreType.DMA((_NBUF,)),
            pltpu.SemaphoreType.DMA((_NBUF,)),
        ],
        compiler_params=pltpu.CompilerParams(use_tc_tiling_on_sc=False),
    )
    def k(idx_hbm, table_hbm, out_hbm, idx_v, rows_v, sem_i, sem_g, sem_o):
        wid = lax.axis_index("s") * _NC + lax.axis_index("c")
        base = wid * _RPW

        def start_idx(s, b):
            pltpu.async_copy(
                idx_hbm.at[pl.ds(base + s * _RPS, _RPS)],
                idx_v.at[b], sem_i.at[b])

        def wait_idx(b):
            pltpu.make_async_copy(
                idx_hbm.at[pl.ds(base, _RPS)], idx_v.at[b],
                sem_i.at[b]).wait()

        def start_gather(b):
            for h in range(_RPS):
                pltpu.async_copy(table_hbm.at[idx_v.at[b, h]],
                                 rows_v.at[b, h], sem_g.at[b])

        def wait_gather(b):
            for h in range(_RPS):
                pltpu.make_async_copy(table_hbm.at[idx_v.at[b, h]],
                                      rows_v.at[b, h], sem_g.at[b]).wait()

        def start_out(s, b):
            pltpu.async_copy(
                rows_v.at[b],
                out_hbm.at[pl.ds(base + s * _RPS, _RPS)], sem_o.at[b])

        def wait_out(b):
            pltpu.make_async_copy(
                rows_v.at[b], out_hbm.at[pl.ds(base, _RPS)],
                sem_o.at[b]).wait()

        for b in range(_NBUF):
            start_idx(b, b)
        wait_idx(0)
        start_gather(0)
        for b in range(1, _NBUF):
            wait_idx(b)
            start_gather(b)
            wait_gather(b - 1)
            start_out(b - 1, b - 1)
            start_idx(b - 1 + _NBUF, b - 1)

        def body(g, carry):
            for b in range(_NBUF):
                s = g * _NBUF + b
                pb = (b - 1) % _NBUF
                wait_out(b)
                wait_idx(b)
                start_gather(b)
                wait_gather(pb)
                start_out(s - 1, pb)

                @pl.when(s - 1 + _NBUF < _STEPS)
                def _():
                    start_idx(s - 1 + _NBUF, pb)
            return carry

        lax.fori_loop(1, _GROUPS, body, 0)

        last = _NBUF - 1
        wait_gather(last)
        start_out(_STEPS - 1, last)
        for b in range(_NBUF):
            wait_out(b)

    return k


_gather = _make_gather()


@jax.jit
def kernel(input_ids, weight):
    ids = input_ids if input_ids.dtype == jnp.int32 else input_ids.astype(jnp.int32)
    return _gather(ids, weight)
